# baseline (speedup 1.0000x reference)
import functools

import jax
import jax.numpy as jnp
from jax import lax
from jax.experimental import pallas as pl
from jax.experimental.pallas import tpu as pltpu
from jax.experimental.pallas import tpu_sc as plsc

N, E, D = 10000, 320000, 128
NC, NS = 2, 16
NW = NC * NS
CHUNK = 64
CP = 160
NBUF = 4
IBUF = 8
E_PAD = NW * CP * CHUNK
N_PAD = 10112
RPT = N_PAD // NS
DEG_W = 16
TC_BLK = 400
SLICES = tuple((o, min(CHUNK, RPT - o)) for o in range(0, RPT, CHUNK))


def _make_sc_agg(with_deg):
    mesh = plsc.VectorSubcoreMesh(core_axis_name="c", subcore_axis_name="s")
    out_type = [jax.ShapeDtypeStruct((NC * N_PAD, D), jnp.float32)]
    scratch = [
        pltpu.VMEM((IBUF, 2, CHUNK), jnp.int32),
        pltpu.VMEM((NBUF, CHUNK, D), jnp.float32),
        pltpu.VMEM_SHARED((N_PAD, D), jnp.float32),
        pltpu.SemaphoreType.DMA((IBUF,)),
        pltpu.SemaphoreType.DMA((NBUF,)),
        pltpu.SemaphoreType.DMA((NBUF,)),
    ]
    if with_deg:
        out_type.append(jax.ShapeDtypeStruct((NC * N_PAD, DEG_W), jnp.float32))
        scratch += [
            pltpu.VMEM((CHUNK, DEG_W), jnp.float32),
            pltpu.VMEM_SHARED((N_PAD, DEG_W), jnp.float32),
            pltpu.SemaphoreType.DMA((NBUF,)),
        ]

    def body(*refs):
        if with_deg:
            (x_hbm, idx_hbm, z_d, z_g, one_g,
             agg_out, deg_out, ibuf, rows_v, acc, isem, gsem, ssem,
             ones_v, dacc, dsem) = refs
        else:
            (x_hbm, idx_hbm, z_d,
             agg_out, ibuf, rows_v, acc, isem, gsem, ssem) = refs
        c = lax.axis_index("c")
        s = lax.axis_index("s")
        wid = c * NS + s
        row0 = s * RPT

        pltpu.sync_copy(z_d, rows_v.at[0])
        for off, sz in SLICES:
            pltpu.sync_copy(rows_v.at[0].at[pl.ds(0, sz)],
                            acc.at[pl.ds(row0 + off, sz)])
        if with_deg:
            pltpu.sync_copy(z_g, ones_v)
            for off, sz in SLICES:
                pltpu.sync_copy(ones_v.at[pl.ds(0, sz)],
                                dacc.at[pl.ds(row0 + off, sz)])
            pltpu.sync_copy(one_g, ones_v)
        plsc.subcore_barrier()

        def idx_fetch(j, k):
            pltpu.async_copy(idx_hbm.at[wid].at[j], ibuf.at[k], isem.at[k])

        def idx_wait(j, k):
            pltpu.make_async_copy(idx_hbm.at[wid].at[j], ibuf.at[k],
                                  isem.at[k]).wait()

        def start_gather(k, b):
            pltpu.async_copy(x_hbm.at[ibuf.at[k].at[0]], rows_v.at[b],
                             gsem.at[b])

        def wait_gather(k, b):
            pltpu.make_async_copy(x_hbm.at[ibuf.at[k].at[0]], rows_v.at[b],
                                  gsem.at[b]).wait()

        def start_scatters(k, b):
            pltpu.async_copy(rows_v.at[b], acc.at[ibuf.at[k].at[1]],
                             ssem.at[b], add=True)
            if with_deg:
                pltpu.async_copy(ones_v, dacc.at[ibuf.at[k].at[1]],
                                 dsem.at[b], add=True)

        def wait_scatters(k, b):
            pltpu.make_async_copy(rows_v.at[b], acc.at[ibuf.at[k].at[1]],
                                  ssem.at[b]).wait()
            if with_deg:
                pltpu.make_async_copy(ones_v, dacc.at[ibuf.at[k].at[1]],
                                      dsem.at[b]).wait()

        for j in range(IBUF):
            idx_fetch(j, j)
        for b in range(NBUF):
            idx_wait(b, b)
            start_gather(b, b)

        @pl.loop(0, CP // NBUF)
        def _(t):
            for b in range(NBUF):
                j = NBUF * t + b
                k = j % IBUF
                wait_gather(k, b)
                start_scatters(k, b)
                jn = j + NBUF

                @pl.when(jn < CP)
                def _():
                    kn = jn % IBUF
                    idx_wait(jn, kn)
                    wait_scatters(k, b)
                    start_gather(kn, b)

                jf = j + IBUF

                @pl.when(jf < CP)
                def _():
                    idx_fetch(jf, k)

        for b in range(NBUF):
            j = CP - NBUF + b
            wait_scatters(j % IBUF, b)

        plsc.subcore_barrier()
        out_row0 = c * N_PAD + row0
        for off, sz in SLICES:
            pltpu.sync_copy(acc.at[pl.ds(row0 + off, sz)],
                            rows_v.at[0].at[pl.ds(0, sz)])
            pltpu.sync_copy(rows_v.at[0].at[pl.ds(0, sz)],
                            agg_out.at[pl.ds(out_row0 + off, sz)])
        if with_deg:
            for off, sz in SLICES:
                pltpu.sync_copy(dacc.at[pl.ds(row0 + off, sz)],
                                ones_v.at[pl.ds(0, sz)])
                pltpu.sync_copy(ones_v.at[pl.ds(0, sz)],
                                deg_out.at[pl.ds(out_row0 + off, sz)])

    return pl.kernel(body, out_type=out_type, mesh=mesh,
                     scratch_types=scratch,
                     compiler_params=pltpu.CompilerParams(
                         use_tc_tiling_on_sc=False))


_sc_agg_deg = _make_sc_agg(True)
_sc_agg = _make_sc_agg(False)


def _tc_body(p0_ref, p1_ref, d0_ref, d1_ref, x_ref, wl_ref, wr_ref, b_ref,
             o_ref, *, relu):
    deg = d0_ref[:, :1] + d1_ref[:, :1]
    deg = jnp.maximum(deg, 1.0)
    mean = (p0_ref[...] + p1_ref[...]) / deg
    acc = jnp.dot(mean, wl_ref[...], preferred_element_type=jnp.float32,
                  precision=lax.Precision.HIGHEST)
    acc = acc + jnp.dot(x_ref[...], wr_ref[...],
                        preferred_element_type=jnp.float32,
                        precision=lax.Precision.HIGHEST)
    acc = acc + b_ref[...]
    if relu:
        acc = jnp.maximum(acc, 0.0)
    o_ref[...] = acc


def _tc_layer(p0, p1, d0, d1, xin, wlT, wrT, b2, relu):
    row_spec = pl.BlockSpec((TC_BLK, D), lambda i: (i, 0))
    deg_spec = pl.BlockSpec((TC_BLK, DEG_W), lambda i: (i, 0))
    w_spec = pl.BlockSpec((D, D), lambda i: (0, 0))
    b_spec = pl.BlockSpec((1, D), lambda i: (0, 0))
    return pl.pallas_call(
        functools.partial(_tc_body, relu=relu),
        grid=(N // TC_BLK,),
        in_specs=[row_spec, row_spec, deg_spec, deg_spec, row_spec,
                  w_spec, w_spec, b_spec],
        out_specs=row_spec,
        out_shape=jax.ShapeDtypeStruct((N, D), jnp.float32),
    )(p0, p1, d0, d1, xin, wlT, wrT, b2)


def kernel(x, edge_index, W_l1, b_l1, W_r1, W_l2, b_l2, W_r2):
    ept = E // NW
    ppt = CP * CHUNK - ept
    src2 = edge_index[0].reshape(NW, ept)
    dst2 = edge_index[1].reshape(NW, ept)
    pad_src = jnp.zeros((NW, ppt), jnp.int32)
    w = jnp.arange(NW, dtype=jnp.int32)[:, None]
    pad_dst = N + (jnp.arange(ppt, dtype=jnp.int32)[None, :] + 7 * w) % (N_PAD - N)
    src3 = jnp.concatenate([src2, pad_src], axis=1).reshape(NW, CP, CHUNK)
    dst3 = jnp.concatenate([dst2, pad_dst], axis=1).reshape(NW, CP, CHUNK)
    idx4 = jnp.stack([src3, dst3], axis=2)

    zeros_d = jnp.zeros((CHUNK, D), jnp.float32)
    zeros_g = jnp.zeros((CHUNK, DEG_W), jnp.float32)
    ones_g = jnp.ones((CHUNK, DEG_W), jnp.float32)

    agg1, deg = _sc_agg_deg(x, idx4, zeros_d, zeros_g, ones_g)
    p0, p1 = agg1[:N], agg1[N_PAD:N_PAD + N]
    d0, d1 = deg[:N], deg[N_PAD:N_PAD + N]
    h = _tc_layer(p0, p1, d0, d1, x, W_l1.T, W_r1.T,
                  b_l1.reshape(1, D), relu=True)

    (agg2,) = _sc_agg(h, idx4, zeros_d)
    q0, q1 = agg2[:N], agg2[N_PAD:N_PAD + N]
    out = _tc_layer(q0, q1, d0, d1, h, W_l2.T, W_r2.T,
                    b_l2.reshape(1, D), relu=False)
    return out


# speedup vs baseline: 1.0031x; 1.0031x over previous
import functools

import jax
import jax.numpy as jnp
from jax import lax
from jax.experimental import pallas as pl
from jax.experimental.pallas import tpu as pltpu
from jax.experimental.pallas import tpu_sc as plsc

N, E, D = 10000, 320000, 128
NC, NS = 2, 16
NW = NC * NS
CHUNK = 64
CP = 160
NBUF = 4
IBUF = 8
E_PAD = NW * CP * CHUNK
N_PAD = 10112
RPT = N_PAD // NS
DEG_W = 16
TC_BLK = 400
SLICES = tuple((o, min(CHUNK, RPT - o)) for o in range(0, RPT, CHUNK))


def _make_sc_agg(with_deg):
    mesh = plsc.VectorSubcoreMesh(core_axis_name="c", subcore_axis_name="s")
    out_type = [jax.ShapeDtypeStruct((NC * N_PAD, D), jnp.float32)]
    scratch = [
        pltpu.VMEM((IBUF, 2, CHUNK), jnp.int32),
        pltpu.VMEM((NBUF, CHUNK, D), jnp.float32),
        pltpu.VMEM_SHARED((N_PAD, D), jnp.float32),
        pltpu.SemaphoreType.DMA((IBUF,)),
        pltpu.SemaphoreType.DMA((NBUF,)),
        pltpu.SemaphoreType.DMA((NBUF,)),
    ]
    if with_deg:
        out_type.append(jax.ShapeDtypeStruct((NC * N_PAD, DEG_W), jnp.float32))
        scratch += [
            pltpu.VMEM((CHUNK, DEG_W), jnp.float32),
            pltpu.VMEM_SHARED((N_PAD, DEG_W), jnp.float32),
            pltpu.SemaphoreType.DMA((NBUF,)),
        ]

    def body(*refs):
        if with_deg:
            (x_hbm, idx_hbm, z_d, z_g, one_g,
             agg_out, deg_out, ibuf, rows_v, acc, isem, gsem, ssem,
             ones_v, dacc, dsem) = refs
        else:
            (x_hbm, idx_hbm, z_d,
             agg_out, ibuf, rows_v, acc, isem, gsem, ssem) = refs
        c = lax.axis_index("c")
        s = lax.axis_index("s")
        wid = c * NS + s
        row0 = s * RPT

        pltpu.sync_copy(z_d, rows_v.at[0])
        for off, sz in SLICES:
            pltpu.sync_copy(rows_v.at[0].at[pl.ds(0, sz)],
                            acc.at[pl.ds(row0 + off, sz)])
        if with_deg:
            pltpu.sync_copy(z_g, ones_v)
            for off, sz in SLICES:
                pltpu.sync_copy(ones_v.at[pl.ds(0, sz)],
                                dacc.at[pl.ds(row0 + off, sz)])
            pltpu.sync_copy(one_g, ones_v)
        plsc.subcore_barrier()

        def idx_fetch(j, k):
            pltpu.async_copy(idx_hbm.at[wid].at[j], ibuf.at[k], isem.at[k])

        def idx_wait(j, k):
            pltpu.make_async_copy(idx_hbm.at[wid].at[j], ibuf.at[k],
                                  isem.at[k]).wait()

        def start_gather(k, b):
            pltpu.async_copy(x_hbm.at[ibuf.at[k].at[0]], rows_v.at[b],
                             gsem.at[b])

        def wait_gather(k, b):
            pltpu.make_async_copy(x_hbm.at[ibuf.at[k].at[0]], rows_v.at[b],
                                  gsem.at[b]).wait()

        def start_scatters(k, b):
            pltpu.async_copy(rows_v.at[b], acc.at[ibuf.at[k].at[1]],
                             ssem.at[b], add=True)
            if with_deg:
                pltpu.async_copy(ones_v, dacc.at[ibuf.at[k].at[1]],
                                 dsem.at[b], add=True)

        def wait_scatters(k, b):
            pltpu.make_async_copy(rows_v.at[b], acc.at[ibuf.at[k].at[1]],
                                  ssem.at[b]).wait()
            if with_deg:
                pltpu.make_async_copy(ones_v, dacc.at[ibuf.at[k].at[1]],
                                      dsem.at[b]).wait()

        for j in range(IBUF):
            idx_fetch(j, j)
        for b in range(NBUF):
            idx_wait(b, b)
            start_gather(b, b)

        @pl.loop(0, CP // NBUF)
        def _(t):
            for b in range(NBUF):
                j = NBUF * t + b
                k = j % IBUF
                wait_gather(k, b)
                start_scatters(k, b)
                jn = j + NBUF

                @pl.when(jn < CP)
                def _():
                    kn = jn % IBUF
                    idx_wait(jn, kn)
                    wait_scatters(k, b)
                    start_gather(kn, b)

                jf = j + IBUF

                @pl.when(jf < CP)
                def _():
                    idx_fetch(jf, k)

        for b in range(NBUF):
            j = CP - NBUF + b
            wait_scatters(j % IBUF, b)

        plsc.subcore_barrier()
        out_row0 = c * N_PAD + row0
        for off, sz in SLICES:
            pltpu.sync_copy(acc.at[pl.ds(row0 + off, sz)],
                            rows_v.at[0].at[pl.ds(0, sz)])
            pltpu.sync_copy(rows_v.at[0].at[pl.ds(0, sz)],
                            agg_out.at[pl.ds(out_row0 + off, sz)])
        if with_deg:
            for off, sz in SLICES:
                pltpu.sync_copy(dacc.at[pl.ds(row0 + off, sz)],
                                ones_v.at[pl.ds(0, sz)])
                pltpu.sync_copy(ones_v.at[pl.ds(0, sz)],
                                deg_out.at[pl.ds(out_row0 + off, sz)])

    return pl.kernel(body, out_type=out_type, mesh=mesh,
                     scratch_types=scratch,
                     compiler_params=pltpu.CompilerParams(
                         use_tc_tiling_on_sc=False))


_sc_agg_deg = _make_sc_agg(True)
_sc_agg = _make_sc_agg(False)


def _tc_xr_body(x_ref, wr_ref, b_ref, o_ref):
    o_ref[...] = jnp.dot(x_ref[...], wr_ref[...],
                         preferred_element_type=jnp.float32,
                         precision=lax.Precision.HIGHEST) + b_ref[...]


def _tc_xr(xin, wrT, b2):
    row_spec = pl.BlockSpec((TC_BLK, D), lambda i: (i, 0))
    return pl.pallas_call(
        _tc_xr_body,
        grid=(N // TC_BLK,),
        in_specs=[row_spec, pl.BlockSpec((D, D), lambda i: (0, 0)),
                  pl.BlockSpec((1, D), lambda i: (0, 0))],
        out_specs=row_spec,
        out_shape=jax.ShapeDtypeStruct((N, D), jnp.float32),
    )(xin, wrT, b2)


def _tc_body(p0_ref, p1_ref, d0_ref, d1_ref, xr_ref, wl_ref, o_ref, *, relu):
    deg = d0_ref[:, :1] + d1_ref[:, :1]
    deg = jnp.maximum(deg, 1.0)
    mean = (p0_ref[...] + p1_ref[...]) / deg
    acc = jnp.dot(mean, wl_ref[...], preferred_element_type=jnp.float32,
                  precision=lax.Precision.HIGHEST)
    acc = acc + xr_ref[...]
    if relu:
        acc = jnp.maximum(acc, 0.0)
    o_ref[...] = acc


def _tc_layer(p0, p1, d0, d1, xr, wlT, relu):
    row_spec = pl.BlockSpec((TC_BLK, D), lambda i: (i, 0))
    deg_spec = pl.BlockSpec((TC_BLK, DEG_W), lambda i: (i, 0))
    w_spec = pl.BlockSpec((D, D), lambda i: (0, 0))
    return pl.pallas_call(
        functools.partial(_tc_body, relu=relu),
        grid=(N // TC_BLK,),
        in_specs=[row_spec, row_spec, deg_spec, deg_spec, row_spec, w_spec],
        out_specs=row_spec,
        out_shape=jax.ShapeDtypeStruct((N, D), jnp.float32),
    )(p0, p1, d0, d1, xr, wlT)


def kernel(x, edge_index, W_l1, b_l1, W_r1, W_l2, b_l2, W_r2):
    ept = E // NW
    ppt = CP * CHUNK - ept
    src2 = edge_index[0].reshape(NW, ept)
    dst2 = edge_index[1].reshape(NW, ept)
    pad_src = jnp.zeros((NW, ppt), jnp.int32)
    w = jnp.arange(NW, dtype=jnp.int32)[:, None]
    pad_dst = N + (jnp.arange(ppt, dtype=jnp.int32)[None, :] + 7 * w) % (N_PAD - N)
    src3 = jnp.concatenate([src2, pad_src], axis=1).reshape(NW, CP, CHUNK)
    dst3 = jnp.concatenate([dst2, pad_dst], axis=1).reshape(NW, CP, CHUNK)
    idx4 = jnp.stack([src3, dst3], axis=2)

    zeros_d = jnp.zeros((CHUNK, D), jnp.float32)
    zeros_g = jnp.zeros((CHUNK, DEG_W), jnp.float32)
    ones_g = jnp.ones((CHUNK, DEG_W), jnp.float32)

    agg1, deg = _sc_agg_deg(x, idx4, zeros_d, zeros_g, ones_g)
    xr1 = _tc_xr(x, W_r1.T, b_l1.reshape(1, D))
    p0, p1 = agg1[:N], agg1[N_PAD:N_PAD + N]
    d0, d1 = deg[:N], deg[N_PAD:N_PAD + N]
    h = _tc_layer(p0, p1, d0, d1, xr1, W_l1.T, relu=True)

    (agg2,) = _sc_agg(h, idx4, zeros_d)
    xr2 = _tc_xr(h, W_r2.T, b_l2.reshape(1, D))
    q0, q1 = agg2[:N], agg2[N_PAD:N_PAD + N]
    out = _tc_layer(q0, q1, d0, d1, xr2, W_l2.T, relu=False)
    return out


# speedup vs baseline: 1.0061x; 1.0030x over previous
import functools

import jax
import jax.numpy as jnp
from jax import lax
from jax.experimental import pallas as pl
from jax.experimental.pallas import tpu as pltpu
from jax.experimental.pallas import tpu_sc as plsc

N, E, D = 10000, 320000, 128
NC, NS = 2, 16
NW = NC * NS
CHUNK = 64
CP = 160
NBUF = 4
IBUF = 8
E_PAD = NW * CP * CHUNK
N_PAD = 10112
RPT = N_PAD // NS
DEG_W = 16
TC_BLK = 400
SLICES = tuple((o, min(CHUNK, RPT - o)) for o in range(0, RPT, CHUNK))


def _make_sc_agg(with_deg):
    mesh = plsc.VectorSubcoreMesh(core_axis_name="c", subcore_axis_name="s")
    out_type = [jax.ShapeDtypeStruct((NC * N_PAD, D), jnp.float32)]
    scratch = [
        pltpu.VMEM((IBUF, 2, CHUNK), jnp.int32),
        pltpu.VMEM((NBUF, CHUNK, D), jnp.float32),
        pltpu.VMEM_SHARED((N_PAD, D), jnp.float32),
        pltpu.SemaphoreType.DMA((IBUF,)),
        pltpu.SemaphoreType.DMA((NBUF,)),
        pltpu.SemaphoreType.DMA((NBUF,)),
    ]
    if with_deg:
        out_type.append(jax.ShapeDtypeStruct((NC * N_PAD, DEG_W), jnp.float32))
        scratch += [
            pltpu.VMEM((CHUNK, DEG_W), jnp.float32),
            pltpu.VMEM_SHARED((N_PAD, DEG_W), jnp.float32),
            pltpu.SemaphoreType.DMA((NBUF,)),
        ]

    def body(*refs):
        if with_deg:
            (x_hbm, idx_hbm, z_d, z_g, one_g,
             agg_out, deg_out, ibuf, rows_v, acc, isem, gsem, ssem,
             ones_v, dacc, dsem) = refs
        else:
            (x_hbm, idx_hbm, z_d,
             agg_out, ibuf, rows_v, acc, isem, gsem, ssem) = refs
        c = lax.axis_index("c")
        s = lax.axis_index("s")
        wid = c * NS + s
        row0 = s * RPT

        pltpu.sync_copy(z_d, rows_v.at[0])
        for off, sz in SLICES:
            pltpu.sync_copy(rows_v.at[0].at[pl.ds(0, sz)],
                            acc.at[pl.ds(row0 + off, sz)])
        if with_deg:
            pltpu.sync_copy(z_g, ones_v)
            for off, sz in SLICES:
                pltpu.sync_copy(ones_v.at[pl.ds(0, sz)],
                                dacc.at[pl.ds(row0 + off, sz)])
            pltpu.sync_copy(one_g, ones_v)
        plsc.subcore_barrier()

        def idx_fetch(j, k):
            pltpu.async_copy(idx_hbm.at[wid].at[j], ibuf.at[k], isem.at[k])

        def idx_wait(j, k):
            pltpu.make_async_copy(idx_hbm.at[wid].at[j], ibuf.at[k],
                                  isem.at[k]).wait()

        def start_gather(k, b):
            pltpu.async_copy(x_hbm.at[ibuf.at[k].at[0]], rows_v.at[b],
                             gsem.at[b])

        def wait_gather(k, b):
            pltpu.make_async_copy(x_hbm.at[ibuf.at[k].at[0]], rows_v.at[b],
                                  gsem.at[b]).wait()

        def start_scatters(k, b):
            pltpu.async_copy(rows_v.at[b], acc.at[ibuf.at[k].at[1]],
                             ssem.at[b], add=True)
            if with_deg:
                pltpu.async_copy(ones_v, dacc.at[ibuf.at[k].at[1]],
                                 dsem.at[b], add=True)

        def wait_scatters(k, b):
            pltpu.make_async_copy(rows_v.at[b], acc.at[ibuf.at[k].at[1]],
                                  ssem.at[b]).wait()
            if with_deg:
                pltpu.make_async_copy(ones_v, dacc.at[ibuf.at[k].at[1]],
                                      dsem.at[b]).wait()

        for j in range(IBUF):
            idx_fetch(j, j)
        for b in range(NBUF):
            idx_wait(b, b)
            start_gather(b, b)

        @pl.loop(0, CP // NBUF)
        def _(t):
            for b in range(NBUF):
                j = NBUF * t + b
                k = j % IBUF
                wait_gather(k, b)
                start_scatters(k, b)
                jn = j + NBUF

                @pl.when(jn < CP)
                def _():
                    kn = jn % IBUF
                    idx_wait(jn, kn)
                    wait_scatters(k, b)
                    start_gather(kn, b)

                jf = j + IBUF

                @pl.when(jf < CP)
                def _():
                    idx_fetch(jf, k)

        for b in range(NBUF):
            j = CP - NBUF + b
            wait_scatters(j % IBUF, b)

        plsc.subcore_barrier()
        out_row0 = c * N_PAD + row0
        pltpu.sync_copy(acc.at[pl.ds(row0, RPT)],
                        agg_out.at[pl.ds(out_row0, RPT)])
        if with_deg:
            pltpu.sync_copy(dacc.at[pl.ds(row0, RPT)],
                            deg_out.at[pl.ds(out_row0, RPT)])

    return pl.kernel(body, out_type=out_type, mesh=mesh,
                     scratch_types=scratch,
                     compiler_params=pltpu.CompilerParams(
                         use_tc_tiling_on_sc=False))


_sc_agg_deg = _make_sc_agg(True)
_sc_agg = _make_sc_agg(False)


def _tc_xr_body(x_ref, wr_ref, b_ref, o_ref):
    o_ref[...] = jnp.dot(x_ref[...], wr_ref[...],
                         preferred_element_type=jnp.float32,
                         precision=lax.Precision.HIGHEST) + b_ref[...]


def _tc_xr(xin, wrT, b2):
    row_spec = pl.BlockSpec((TC_BLK, D), lambda i: (i, 0))
    return pl.pallas_call(
        _tc_xr_body,
        grid=(N // TC_BLK,),
        in_specs=[row_spec, pl.BlockSpec((D, D), lambda i: (0, 0)),
                  pl.BlockSpec((1, D), lambda i: (0, 0))],
        out_specs=row_spec,
        out_shape=jax.ShapeDtypeStruct((N, D), jnp.float32),
    )(xin, wrT, b2)


def _tc_body(p0_ref, p1_ref, d0_ref, d1_ref, xr_ref, wl_ref, o_ref, *, relu):
    deg = d0_ref[:, :1] + d1_ref[:, :1]
    deg = jnp.maximum(deg, 1.0)
    mean = (p0_ref[...] + p1_ref[...]) / deg
    acc = jnp.dot(mean, wl_ref[...], preferred_element_type=jnp.float32,
                  precision=lax.Precision.HIGHEST)
    acc = acc + xr_ref[...]
    if relu:
        acc = jnp.maximum(acc, 0.0)
    o_ref[...] = acc


def _tc_layer(p0, p1, d0, d1, xr, wlT, relu):
    row_spec = pl.BlockSpec((TC_BLK, D), lambda i: (i, 0))
    deg_spec = pl.BlockSpec((TC_BLK, DEG_W), lambda i: (i, 0))
    w_spec = pl.BlockSpec((D, D), lambda i: (0, 0))
    return pl.pallas_call(
        functools.partial(_tc_body, relu=relu),
        grid=(N // TC_BLK,),
        in_specs=[row_spec, row_spec, deg_spec, deg_spec, row_spec, w_spec],
        out_specs=row_spec,
        out_shape=jax.ShapeDtypeStruct((N, D), jnp.float32),
    )(p0, p1, d0, d1, xr, wlT)


def kernel(x, edge_index, W_l1, b_l1, W_r1, W_l2, b_l2, W_r2):
    ept = E // NW
    ppt = CP * CHUNK - ept
    src2 = edge_index[0].reshape(NW, ept)
    dst2 = edge_index[1].reshape(NW, ept)
    pad_src = jnp.zeros((NW, ppt), jnp.int32)
    w = jnp.arange(NW, dtype=jnp.int32)[:, None]
    pad_dst = N + (jnp.arange(ppt, dtype=jnp.int32)[None, :] + 7 * w) % (N_PAD - N)
    src3 = jnp.concatenate([src2, pad_src], axis=1).reshape(NW, CP, CHUNK)
    dst3 = jnp.concatenate([dst2, pad_dst], axis=1).reshape(NW, CP, CHUNK)
    idx4 = jnp.stack([src3, dst3], axis=2)

    zeros_d = jnp.zeros((CHUNK, D), jnp.float32)
    zeros_g = jnp.zeros((CHUNK, DEG_W), jnp.float32)
    ones_g = jnp.ones((CHUNK, DEG_W), jnp.float32)

    agg1, deg = _sc_agg_deg(x, idx4, zeros_d, zeros_g, ones_g)
    xr1 = _tc_xr(x, W_r1.T, b_l1.reshape(1, D))
    p0, p1 = agg1[:N], agg1[N_PAD:N_PAD + N]
    d0, d1 = deg[:N], deg[N_PAD:N_PAD + N]
    h = _tc_layer(p0, p1, d0, d1, xr1, W_l1.T, relu=True)

    (agg2,) = _sc_agg(h, idx4, zeros_d)
    xr2 = _tc_xr(h, W_r2.T, b_l2.reshape(1, D))
    q0, q1 = agg2[:N], agg2[N_PAD:N_PAD + N]
    out = _tc_layer(q0, q1, d0, d1, xr2, W_l2.T, relu=False)
    return out
